# baseline (speedup 1.0000x reference)
import functools

import jax
import jax.numpy as jnp
from jax.experimental import pallas as pl
from jax.experimental.pallas import tpu as pltpu

N = 10000
F = 512
BM = 200


def _dot(a, b):
    return jnp.dot(a, b, preferred_element_type=jnp.float32)


def _small_mm_body(x_ref, w_ref, o_ref):
    o_ref[...] = _dot(x_ref[...], w_ref[...]).astype(jnp.bfloat16)


def _layer_body(adj_ref, s_ref, b_ref, w_ref, o_ref, *, fuse_w):
    a = adj_ref[...].astype(jnp.bfloat16)
    acc = _dot(a, s_ref[...])
    act = jnp.tanh(acc + b_ref[...])
    if fuse_w:
        o_ref[...] = _dot(act, w_ref[...]).astype(jnp.bfloat16)
    else:
        o_ref[...] = act


def _layer(adj, s, b2d, w, fuse_w):
    body = functools.partial(_layer_body, fuse_w=fuse_w)
    out_dtype = jnp.bfloat16 if fuse_w else jnp.float32
    return pl.pallas_call(
        body,
        grid=(N // BM,),
        in_specs=[
            pl.BlockSpec((BM, N), lambda i: (i, 0)),
            pl.BlockSpec((N, F), lambda i: (0, 0)),
            pl.BlockSpec((1, F), lambda i: (0, 0)),
            pl.BlockSpec((F, F), lambda i: (0, 0)),
        ],
        out_specs=pl.BlockSpec((BM, F), lambda i: (i, 0)),
        out_shape=jax.ShapeDtypeStruct((N, F), out_dtype),
        compiler_params=pltpu.CompilerParams(
            dimension_semantics=("arbitrary",),
        ),
    )(adj, s, b2d, w)


def kernel(x, adj, W1, b1, W2, b2):
    s1 = pl.pallas_call(
        _small_mm_body,
        grid=(N // BM,),
        in_specs=[
            pl.BlockSpec((BM, F), lambda i: (i, 0)),
            pl.BlockSpec((F, F), lambda i: (0, 0)),
        ],
        out_specs=pl.BlockSpec((BM, F), lambda i: (i, 0)),
        out_shape=jax.ShapeDtypeStruct((N, F), jnp.bfloat16),
        compiler_params=pltpu.CompilerParams(
            dimension_semantics=("arbitrary",),
        ),
    )(x, W1)
    s2 = _layer(adj, s1, b1.reshape(1, F), W2, fuse_w=True)
    out = _layer(adj, s2, b2.reshape(1, F), W2, fuse_w=False)
    return out


# speedup vs baseline: 1.1466x; 1.1466x over previous
import functools

import jax
import jax.numpy as jnp
from jax.experimental import pallas as pl
from jax.experimental.pallas import tpu as pltpu

N = 10000
F = 512
BM = 400


def _dot(a, b):
    return jnp.dot(a, b, preferred_element_type=jnp.float32)


def _small_mm_body(x_ref, w_ref, o_ref):
    o_ref[...] = _dot(x_ref[...], w_ref[...]).astype(jnp.bfloat16)


def _layer_body(adj_ref, s_ref, b_ref, w_ref, o_ref, *, fuse_w):
    acc = jax.lax.dot_general(
        adj_ref[...], s_ref[...],
        dimension_numbers=(((1,), (0,)), ((), ())),
        preferred_element_type=jnp.float32,
    )
    act = jnp.tanh(acc + b_ref[...])
    if fuse_w:
        o_ref[...] = _dot(act, w_ref[...]).astype(jnp.bfloat16)
    else:
        o_ref[...] = act


def _layer(adj, s, b2d, w, fuse_w):
    body = functools.partial(_layer_body, fuse_w=fuse_w)
    out_dtype = jnp.bfloat16 if fuse_w else jnp.float32
    return pl.pallas_call(
        body,
        grid=(N // BM,),
        in_specs=[
            pl.BlockSpec((BM, N), lambda i: (i, 0)),
            pl.BlockSpec((N, F), lambda i: (0, 0)),
            pl.BlockSpec((1, F), lambda i: (0, 0)),
            pl.BlockSpec((F, F), lambda i: (0, 0)),
        ],
        out_specs=pl.BlockSpec((BM, F), lambda i: (i, 0)),
        out_shape=jax.ShapeDtypeStruct((N, F), out_dtype),
        compiler_params=pltpu.CompilerParams(
            dimension_semantics=("arbitrary",),
        ),
    )(adj, s, b2d, w)


def kernel(x, adj, W1, b1, W2, b2):
    s1 = pl.pallas_call(
        _small_mm_body,
        grid=(N // BM,),
        in_specs=[
            pl.BlockSpec((BM, F), lambda i: (i, 0)),
            pl.BlockSpec((F, F), lambda i: (0, 0)),
        ],
        out_specs=pl.BlockSpec((BM, F), lambda i: (i, 0)),
        out_shape=jax.ShapeDtypeStruct((N, F), jnp.bfloat16),
        compiler_params=pltpu.CompilerParams(
            dimension_semantics=("arbitrary",),
        ),
    )(x, W1)
    s2 = _layer(adj, s1, b1.reshape(1, F), W2, fuse_w=True)
    out = _layer(adj, s2, b2.reshape(1, F), W2, fuse_w=False)
    return out


# speedup vs baseline: 1.1477x; 1.0010x over previous
import functools

import jax
import jax.numpy as jnp
from jax.experimental import pallas as pl
from jax.experimental.pallas import tpu as pltpu

N = 10000
F = 512
BM = 400


def _dot(a, b):
    return jnp.dot(a, b, preferred_element_type=jnp.float32)


def _small_mm_body(x_ref, w_ref, o_ref):
    o_ref[...] = _dot(x_ref[...], w_ref[...]).astype(jnp.bfloat16)


def _layer_body(adj_ref, s_ref, b_ref, w_ref, o_ref, *, fuse_w):
    acc = jax.lax.dot_general(
        adj_ref[...], s_ref[...],
        dimension_numbers=(((1,), (0,)), ((), ())),
        preferred_element_type=jnp.float32,
    )
    act = jnp.tanh(acc + b_ref[...])
    if fuse_w:
        o_ref[...] = _dot(act, w_ref[...]).astype(jnp.bfloat16)
    else:
        o_ref[...] = act


def _layer(adj, s, b2d, w, fuse_w):
    body = functools.partial(_layer_body, fuse_w=fuse_w)
    out_dtype = jnp.bfloat16 if fuse_w else jnp.float32
    return pl.pallas_call(
        body,
        grid=(N // BM,),
        in_specs=[
            pl.BlockSpec((BM, N), lambda i: (i, 0)),
            pl.BlockSpec((N, F), lambda i: (0, 0)),
            pl.BlockSpec((1, F), lambda i: (0, 0)),
            pl.BlockSpec((F, F), lambda i: (0, 0)),
        ],
        out_specs=pl.BlockSpec((BM, F), lambda i: (i, 0)),
        out_shape=jax.ShapeDtypeStruct((N, F), out_dtype),
        compiler_params=pltpu.CompilerParams(
            dimension_semantics=("arbitrary",),
            vmem_limit_bytes=120 * 1024 * 1024,
        ),
    )(adj, s, b2d, w)


def kernel(x, adj, W1, b1, W2, b2):
    s1 = pl.pallas_call(
        _small_mm_body,
        grid=(N // BM,),
        in_specs=[
            pl.BlockSpec((BM, F), lambda i: (i, 0)),
            pl.BlockSpec((F, F), lambda i: (0, 0)),
        ],
        out_specs=pl.BlockSpec((BM, F), lambda i: (i, 0)),
        out_shape=jax.ShapeDtypeStruct((N, F), jnp.bfloat16),
        compiler_params=pltpu.CompilerParams(
            dimension_semantics=("arbitrary",),
        ),
    )(x, W1)
    s2 = _layer(adj, s1, b1.reshape(1, F), W2, fuse_w=True)
    out = _layer(adj, s2, b2.reshape(1, F), W2, fuse_w=False)
    return out


# speedup vs baseline: 1.1923x; 1.0389x over previous
import jax
import jax.numpy as jnp
from jax.experimental import pallas as pl
from jax.experimental.pallas import tpu as pltpu

N = 10000
F = 512
BM = 400
NB = N // BM


def _dot(a, b):
    return jnp.dot(a, b, preferred_element_type=jnp.float32)


def _mixed_dot(a, b):
    return jax.lax.dot_general(
        a, b,
        dimension_numbers=(((1,), (0,)), ((), ())),
        preferred_element_type=jnp.float32,
    )


def _small_mm_body(x_ref, w_ref, o_ref):
    o_ref[...] = _dot(x_ref[...], w_ref[...]).astype(jnp.bfloat16)


def _fused_body(s1_ref, adj_ref, b1_ref, b2_ref, w2_ref, o_ref, s2_ref):
    i = pl.program_id(0)

    @pl.when(i < NB)
    def _layer1():
        acc = _mixed_dot(adj_ref[...], s1_ref[...])
        act = jnp.tanh(acc + b1_ref[...])
        s2_ref[pl.ds((i % NB) * BM, BM), :] = _dot(act, w2_ref[...]).astype(
            jnp.bfloat16)

    @pl.when(i >= NB)
    def _layer2():
        acc = _mixed_dot(adj_ref[...], s2_ref[...])
        o_ref[...] = jnp.tanh(acc + b2_ref[...])


def kernel(x, adj, W1, b1, W2, b2):
    s1 = pl.pallas_call(
        _small_mm_body,
        grid=(NB,),
        in_specs=[
            pl.BlockSpec((BM, F), lambda i: (i, 0)),
            pl.BlockSpec((F, F), lambda i: (0, 0)),
        ],
        out_specs=pl.BlockSpec((BM, F), lambda i: (i, 0)),
        out_shape=jax.ShapeDtypeStruct((N, F), jnp.bfloat16),
        compiler_params=pltpu.CompilerParams(
            dimension_semantics=("arbitrary",),
        ),
    )(x, W1)
    return pl.pallas_call(
        _fused_body,
        grid=(2 * NB,),
        in_specs=[
            pl.BlockSpec((N, F), lambda i: (0, 0)),
            pl.BlockSpec((BM, N), lambda i: (i % NB, 0)),
            pl.BlockSpec((1, F), lambda i: (0, 0)),
            pl.BlockSpec((1, F), lambda i: (0, 0)),
            pl.BlockSpec((F, F), lambda i: (0, 0)),
        ],
        out_specs=pl.BlockSpec(
            (BM, F), lambda i: (jnp.maximum(i - NB, 0), 0)),
        out_shape=jax.ShapeDtypeStruct((N, F), jnp.float32),
        scratch_shapes=[pltpu.VMEM((N, F), jnp.bfloat16)],
        compiler_params=pltpu.CompilerParams(
            dimension_semantics=("arbitrary",),
        ),
    )(s1, adj, b1.reshape(1, F), b2.reshape(1, F), W2)


# speedup vs baseline: 1.2224x; 1.0252x over previous
import jax
import jax.numpy as jnp
from jax.experimental import pallas as pl
from jax.experimental.pallas import tpu as pltpu

N = 10000
F = 512
BM = 400
NB = N // BM
BX = 400
NS = N // BX


def _dot(a, b):
    return jnp.dot(a, b, preferred_element_type=jnp.float32)


def _mixed_dot(a, b):
    return jax.lax.dot_general(
        a, b,
        dimension_numbers=(((1,), (0,)), ((), ())),
        preferred_element_type=jnp.float32,
    )


def _body(x_ref, adj_ref, w1_ref, b1_ref, b2_ref, w2_ref,
          o_ref, s1_ref, s2_ref):
    i = pl.program_id(0)

    @pl.when(i < NS)
    def _small_mm():
        s1_ref[pl.ds(i * BX, BX), :] = _dot(
            x_ref[...], w1_ref[...]).astype(jnp.bfloat16)

    @pl.when(jnp.logical_and(i >= NS, i < NS + NB))
    def _layer1():
        acc = _mixed_dot(adj_ref[...], s1_ref[...])
        act = jnp.tanh(acc + b1_ref[...])
        s2_ref[pl.ds(((i - NS) % NB) * BM, BM), :] = _dot(
            act, w2_ref[...]).astype(jnp.bfloat16)

    @pl.when(i >= NS + NB)
    def _layer2():
        acc = _mixed_dot(adj_ref[...], s2_ref[...])
        o_ref[...] = jnp.tanh(acc + b2_ref[...])


def kernel(x, adj, W1, b1, W2, b2):
    return pl.pallas_call(
        _body,
        grid=(NS + 2 * NB,),
        in_specs=[
            pl.BlockSpec((BX, F), lambda i: (jnp.minimum(i, NS - 1), 0)),
            pl.BlockSpec((BM, N),
                         lambda i: (jnp.maximum(i - NS, 0) % NB, 0)),
            pl.BlockSpec((F, F), lambda i: (0, 0)),
            pl.BlockSpec((1, F), lambda i: (0, 0)),
            pl.BlockSpec((1, F), lambda i: (0, 0)),
            pl.BlockSpec((F, F), lambda i: (0, 0)),
        ],
        out_specs=pl.BlockSpec(
            (BM, F), lambda i: (jnp.maximum(i - (NS + NB), 0), 0)),
        out_shape=jax.ShapeDtypeStruct((N, F), jnp.float32),
        scratch_shapes=[
            pltpu.VMEM((N, F), jnp.bfloat16),
            pltpu.VMEM((N, F), jnp.bfloat16),
        ],
        compiler_params=pltpu.CompilerParams(
            dimension_semantics=("arbitrary",),
            vmem_limit_bytes=64 * 1024 * 1024,
        ),
    )(x, adj, W1, b1.reshape(1, F), b2.reshape(1, F), W2)


# speedup vs baseline: 1.2679x; 1.0372x over previous
import jax
import jax.numpy as jnp
from jax.experimental import pallas as pl
from jax.experimental.pallas import tpu as pltpu

N = 10000
F = 512
BM = 400
NB = N // BM
BX = 2000
NS = N // BX


def _dot(a, b):
    return jnp.dot(a, b, preferred_element_type=jnp.float32)


def _mixed_dot(a, b):
    return jax.lax.dot_general(
        a, b,
        dimension_numbers=(((1,), (0,)), ((), ())),
        preferred_element_type=jnp.float32,
    )


def _body(x_ref, adj_ref, w1_ref, b1_ref, b2_ref, w2_ref,
          o_ref, s1_ref, s2_ref):
    i = pl.program_id(0)

    @pl.when(i < NS)
    def _small_mm():
        s1_ref[pl.ds(i * BX, BX), :] = _dot(
            x_ref[...], w1_ref[...]).astype(jnp.bfloat16)

    @pl.when(jnp.logical_and(i >= NS, i < NS + NB))
    def _layer1():
        acc = _mixed_dot(adj_ref[...], s1_ref[...])
        act = jnp.tanh(acc + b1_ref[...])
        s2_ref[pl.ds(((i - NS) % NB) * BM, BM), :] = _dot(
            act, w2_ref[...]).astype(jnp.bfloat16)

    @pl.when(i >= NS + NB)
    def _layer2():
        acc = _mixed_dot(adj_ref[...], s2_ref[...])
        o_ref[...] = jnp.tanh(acc + b2_ref[...])


def kernel(x, adj, W1, b1, W2, b2):
    return pl.pallas_call(
        _body,
        grid=(NS + 2 * NB,),
        in_specs=[
            pl.BlockSpec((BX, F), lambda i: (jnp.minimum(i, NS - 1), 0)),
            pl.BlockSpec((BM, N),
                         lambda i: (jnp.maximum(i - NS, 0) % NB, 0)),
            pl.BlockSpec((F, F), lambda i: (0, 0)),
            pl.BlockSpec((1, F), lambda i: (0, 0)),
            pl.BlockSpec((1, F), lambda i: (0, 0)),
            pl.BlockSpec((F, F), lambda i: (0, 0)),
        ],
        out_specs=pl.BlockSpec(
            (BM, F), lambda i: (jnp.maximum(i - (NS + NB), 0), 0)),
        out_shape=jax.ShapeDtypeStruct((N, F), jnp.float32),
        scratch_shapes=[
            pltpu.VMEM((N, F), jnp.bfloat16),
            pltpu.VMEM((N, F), jnp.bfloat16),
        ],
        compiler_params=pltpu.CompilerParams(
            dimension_semantics=("arbitrary",),
            vmem_limit_bytes=64 * 1024 * 1024,
        ),
    )(x, adj, W1, b1.reshape(1, F), b2.reshape(1, F), W2)
